# Initial kernel scaffold; baseline (speedup 1.0000x reference)
import jax
import jax.numpy as jnp
from jax.experimental import pallas as pl


def kernel(node_feats_in, adj_matrix, W, b, a):
    raise NotImplementedError("write your pallas kernel here")


# speedup vs baseline: 24.4492x; 24.4492x over previous
import jax
import jax.numpy as jnp
from jax.experimental import pallas as pl
from jax.experimental.pallas import tpu as pltpu

N = 1024
C_IN = 128
C_OUT = 64
ALPHA = 0.2
NEG = -9e15


def _gat_kernel(x_ref, adj_ref, w_ref, b_ref, a_ref, o_ref):
    x = x_ref[...]
    w = w_ref[...]
    b = b_ref[...]
    a = a_ref[...]

    h = jax.lax.dot_general(x, w, (((1,), (1,)), ((), ())),
                            preferred_element_type=jnp.float32) + b

    a1 = a[:, :C_OUT]
    a2 = a[:, C_OUT:]
    s_col = jax.lax.dot_general(h, a1, (((1,), (1,)), ((), ())),
                                preferred_element_type=jnp.float32)
    t_row = jax.lax.dot_general(a2, h, (((1,), (1,)), ((), ())),
                                preferred_element_type=jnp.float32)

    logits = s_col + t_row
    logits = jnp.where(logits >= 0, logits, ALPHA * logits)
    masked = jnp.where(adj_ref[...] != 0, logits, NEG)
    m = jnp.max(masked, axis=1, keepdims=True)
    e = jnp.exp(masked - m)
    ssum = jnp.sum(e, axis=1, keepdims=True)
    p = e / ssum
    o_ref[...] = jax.lax.dot_general(p, h, (((1,), (0,)), ((), ())),
                                     preferred_element_type=jnp.float32)


def kernel(node_feats_in, adj_matrix, W, b, a):
    x = node_feats_in.reshape(N, C_IN)
    adj = adj_matrix.reshape(N, N)
    b2 = b.reshape(1, C_OUT)
    out = pl.pallas_call(
        _gat_kernel,
        out_shape=jax.ShapeDtypeStruct((N, C_OUT), jnp.float32),
    )(x, adj, W, b2, a)
    return out.reshape(1, N, C_OUT)
